# speedup vs baseline: 3.4917x; 3.4917x over previous
import jax
import jax.numpy as jnp
from jax.experimental import pallas as pl
from jax.experimental.pallas import tpu as pltpu

H = 12
DH = 64
L = 2048
D = 768
K = 16
G = 8
LP = 2176
TLA = 128
TL = 128
SCALE = DH ** -0.5


def _proj_body(x_ref, wq_ref, wk_ref, wv_ref, q_ref, k_ref, v_ref):
    x = x_ref[0]
    q_ref[0] = jnp.dot(x, wq_ref[...], preferred_element_type=jnp.float32)
    k_ref[0] = jnp.dot(x, wk_ref[...], preferred_element_type=jnp.float32)
    v_ref[0] = jnp.dot(x, wv_ref[...], preferred_element_type=jnp.float32)


def _attn_body(idx_ref, q_ref, kp_ref, vp_ref, dist_ref, par_ref, wout_ref,
               bout_ref, o_ref, kg_ref, vg_ref):
    def gather_row(r, carry):
        base = r * K
        for c in range(K):
            row = idx_ref[0, r, c]
            kg_ref[pl.ds(base + c, 1), :] = kp_ref[0, pl.ds(row, 1), :]
            vg_ref[pl.ds(base + c, 1), :] = vp_ref[0, pl.ds(row, 1), :]
        return carry

    jax.lax.fori_loop(0, TL, gather_row, 0)

    q = q_ref[0] * SCALE
    kg = kg_ref[...]
    vg = vg_ref[...]

    d_over = jax.lax.broadcasted_iota(jnp.int32, (D, H), 0) // DH
    h_ids = jax.lax.broadcasted_iota(jnp.int32, (D, H), 1)
    S = (d_over == h_ids).astype(jnp.float32)
    d_over_t = jax.lax.broadcasted_iota(jnp.int32, (H, D), 1) // DH
    h_ids_t = jax.lax.broadcasted_iota(jnp.int32, (H, D), 0)
    St = (d_over_t == h_ids_t).astype(jnp.float32)

    l0 = pl.program_id(1) * TL
    kself = kp_ref[0, pl.ds(l0, TL), :]
    vself = vp_ref[0, pl.ds(l0, TL), :]
    gk = kp_ref[0, pl.ds(L, G), :]
    gv = vp_ref[0, pl.ds(L, G), :]

    qexp_n = jnp.broadcast_to(q[:, None, :], (TL, K, D)).reshape(TL * K, D)
    qexp_g = jnp.broadcast_to(q[:, None, :], (TL, G, D)).reshape(TL * G, D)
    gk_exp = jnp.broadcast_to(gk[None, :, :], (TL, G, D)).reshape(TL * G, D)
    gv_exp = jnp.broadcast_to(gv[None, :, :], (TL, G, D)).reshape(TL * G, D)

    s_n = jnp.dot(qexp_n * kg, S, preferred_element_type=jnp.float32)
    s_s = jnp.dot(q * kself, S, preferred_element_type=jnp.float32)
    s_g = jnp.dot(qexp_g * gk_exp, S, preferred_element_type=jnp.float32)

    inv2s = par_ref[0:1, 0:H].reshape(1, 1, H)
    gbias = par_ref[1:2, 0:1].reshape(1, 1, 1)
    dist = dist_ref[0]
    s_n3 = s_n.reshape(TL, K, H) + (dist * dist)[:, :, None] * inv2s
    s_g3 = s_g.reshape(TL, G, H) + gbias

    m = jnp.maximum(jnp.maximum(s_s, s_n3.max(axis=1)), s_g3.max(axis=1))
    e_s = jnp.exp(s_s - m)
    e_n = jnp.exp(s_n3 - m[:, None, :])
    e_g = jnp.exp(s_g3 - m[:, None, :])
    inv_den = 1.0 / (e_s + e_n.sum(axis=1) + e_g.sum(axis=1))
    p_s = e_s * inv_den
    p_n = e_n * inv_den[:, None, :]
    p_g = e_g * inv_den[:, None, :]

    out = jnp.dot(p_s, St, preferred_element_type=jnp.float32) * vself
    out += (jnp.dot(p_n.reshape(TL * K, H), St,
                    preferred_element_type=jnp.float32) * vg
            ).reshape(TL, K, D).sum(axis=1)
    out += (jnp.dot(p_g.reshape(TL * G, H), St,
                    preferred_element_type=jnp.float32) * gv_exp
            ).reshape(TL, G, D).sum(axis=1)

    o_ref[0] = (jnp.dot(out, wout_ref[...], preferred_element_type=jnp.float32)
                + bout_ref[0:1, :])


def kernel(spatial, topk_indices, rpe, self_rpe, distances, global_latents,
           Wq, Wk, Wv, Wout, b_out, log_sigma, global_bias):
    B = spatial.shape[0]
    xall = jnp.concatenate(
        [spatial, global_latents,
         jnp.zeros((B, LP - L - G, D), spatial.dtype)], axis=1)

    q_all, kp, vp = pl.pallas_call(
        _proj_body,
        grid=(B, LP // TLA),
        in_specs=[
            pl.BlockSpec((1, TLA, D), lambda b, i: (b, i, 0)),
            pl.BlockSpec((D, D), lambda b, i: (0, 0)),
            pl.BlockSpec((D, D), lambda b, i: (0, 0)),
            pl.BlockSpec((D, D), lambda b, i: (0, 0)),
        ],
        out_specs=[
            pl.BlockSpec((1, TLA, D), lambda b, i: (b, i, 0)),
            pl.BlockSpec((1, TLA, D), lambda b, i: (b, i, 0)),
            pl.BlockSpec((1, TLA, D), lambda b, i: (b, i, 0)),
        ],
        out_shape=[jax.ShapeDtypeStruct((B, LP, D), jnp.float32)] * 3,
    )(xall, Wq, Wk, Wv)

    q = q_all[:, :L]
    idx = topk_indices.astype(jnp.int32)
    params = jnp.zeros((8, 128), jnp.float32)
    params = params.at[0, :H].set(-0.5 * jnp.exp(-2.0 * log_sigma))
    params = params.at[1, 0].set(global_bias)
    bout8 = jnp.broadcast_to(b_out[None, :], (8, D))

    out = pl.pallas_call(
        _attn_body,
        grid=(B, L // TL),
        in_specs=[
            pl.BlockSpec((1, TL, K), lambda b, i: (b, i, 0),
                         memory_space=pltpu.SMEM),
            pl.BlockSpec((1, TL, D), lambda b, i: (b, i, 0)),
            pl.BlockSpec((1, LP, D), lambda b, i: (b, 0, 0)),
            pl.BlockSpec((1, LP, D), lambda b, i: (b, 0, 0)),
            pl.BlockSpec((1, TL, K), lambda b, i: (b, i, 0)),
            pl.BlockSpec((8, 128), lambda b, i: (0, 0)),
            pl.BlockSpec((D, D), lambda b, i: (0, 0)),
            pl.BlockSpec((8, D), lambda b, i: (0, 0)),
        ],
        out_specs=pl.BlockSpec((1, TL, D), lambda b, i: (b, i, 0)),
        out_shape=jax.ShapeDtypeStruct((B, L, D), jnp.float32),
        scratch_shapes=[
            pltpu.VMEM((TL * K, D), jnp.float32),
            pltpu.VMEM((TL * K, D), jnp.float32),
        ],
    )(idx, q, kp, vp, distances, params, Wout, bout8)
    return out


# speedup vs baseline: 3.7610x; 1.0771x over previous
import functools
import jax
import jax.numpy as jnp
from jax import lax
from jax.experimental import pallas as pl
from jax.experimental.pallas import tpu as pltpu
from jax.experimental.pallas import tpu_sc as plsc

H = 12
DH = 64
L = 2048
D = 768
K = 16
G = 8
B = 2
LP = 2176
TLA = 128
TL = 128
SCALE = DH ** -0.5

NW = 32
ROWS_PER_W = B * L // NW
CH = 2
IDX_CH = CH * K
NCHUNK = ROWS_PER_W // CH
NBUF = 2


def _proj_body(x_ref, wq_ref, wk_ref, wv_ref, q_ref, k_ref, v_ref):
    x = x_ref[0]
    q_ref[0] = jnp.dot(x, wq_ref[...], preferred_element_type=jnp.float32)
    k_ref[0] = jnp.dot(x, wk_ref[...], preferred_element_type=jnp.float32)
    v_ref[0] = jnp.dot(x, wv_ref[...], preferred_element_type=jnp.float32)


def _sc_gather_body(kp_hbm, vp_hbm, idx_hbm, kg_hbm, vg_hbm,
                    idx_v, kbuf, vbuf, ks0, ks1, vs0, vs1):
    ksem = (ks0, ks1)
    vsem = (vs0, vs1)
    wid = lax.axis_index("s") * 2 + lax.axis_index("c")
    row0 = wid * ROWS_PER_W
    boff = (row0 // L) * LP
    chunk0 = wid * NCHUNK

    pltpu.sync_copy(idx_hbm.at[pl.ds(chunk0, NCHUNK)], idx_v)

    def adjust(c, carry):
        for h in range(IDX_CH // 16):
            sl = pl.ds(h * 16, 16)
            idx_v[c, sl] = idx_v[c, sl] + boff
        return carry

    lax.fori_loop(0, NCHUNK, adjust, 0)

    def group(c, carry):
        pltpu.async_copy(kp_hbm.at[idx_v.at[c]], kbuf.at[0], ksem[0])
        pltpu.async_copy(vp_hbm.at[idx_v.at[c]], vbuf.at[0], vsem[0])
        pltpu.make_async_copy(kp_hbm.at[idx_v.at[c]], kbuf.at[0],
                              ksem[0]).wait()
        pltpu.make_async_copy(vp_hbm.at[idx_v.at[c]], vbuf.at[0],
                              vsem[0]).wait()
        out_off = (chunk0 + c) * IDX_CH
        pltpu.sync_copy(kbuf.at[0], kg_hbm.at[pl.ds(out_off, IDX_CH)])
        pltpu.sync_copy(vbuf.at[0], vg_hbm.at[pl.ds(out_off, IDX_CH)])
        return carry

    lax.fori_loop(0, NCHUNK, group, 0)


def _attn_body(q_ref, kg_ref, vg_ref, kself_ref, vself_ref, gk_ref, gv_ref,
               dist_ref, par_ref, wout_ref, bout_ref, o_ref):
    q = q_ref[0] * SCALE
    kg = kg_ref[0]
    vg = vg_ref[0]

    d_over = lax.broadcasted_iota(jnp.int32, (D, H), 0) // DH
    h_ids = lax.broadcasted_iota(jnp.int32, (D, H), 1)
    S = (d_over == h_ids).astype(jnp.float32)
    d_over_t = lax.broadcasted_iota(jnp.int32, (H, D), 1) // DH
    h_ids_t = lax.broadcasted_iota(jnp.int32, (H, D), 0)
    St = (d_over_t == h_ids_t).astype(jnp.float32)

    kself = kself_ref[0]
    vself = vself_ref[0]
    gk = gk_ref[0]
    gv = gv_ref[0]

    qexp_n = jnp.broadcast_to(q[:, None, :], (TL, K, D)).reshape(TL * K, D)
    qexp_g = jnp.broadcast_to(q[:, None, :], (TL, G, D)).reshape(TL * G, D)
    gk_exp = jnp.broadcast_to(gk[None, :, :], (TL, G, D)).reshape(TL * G, D)
    gv_exp = jnp.broadcast_to(gv[None, :, :], (TL, G, D)).reshape(TL * G, D)

    s_n = jnp.dot(qexp_n * kg, S, preferred_element_type=jnp.float32)
    s_s = jnp.dot(q * kself, S, preferred_element_type=jnp.float32)
    s_g = jnp.dot(qexp_g * gk_exp, S, preferred_element_type=jnp.float32)

    inv2s = par_ref[0:1, 0:H].reshape(1, 1, H)
    gbias = par_ref[1:2, 0:1].reshape(1, 1, 1)
    dist = dist_ref[0]
    s_n3 = s_n.reshape(TL, K, H) + (dist * dist)[:, :, None] * inv2s
    s_g3 = s_g.reshape(TL, G, H) + gbias

    m = jnp.maximum(jnp.maximum(s_s, s_n3.max(axis=1)), s_g3.max(axis=1))
    e_s = jnp.exp(s_s - m)
    e_n = jnp.exp(s_n3 - m[:, None, :])
    e_g = jnp.exp(s_g3 - m[:, None, :])
    inv_den = 1.0 / (e_s + e_n.sum(axis=1) + e_g.sum(axis=1))
    p_s = e_s * inv_den
    p_n = e_n * inv_den[:, None, :]
    p_g = e_g * inv_den[:, None, :]

    out = jnp.dot(p_s, St, preferred_element_type=jnp.float32) * vself
    out += (jnp.dot(p_n.reshape(TL * K, H), St,
                    preferred_element_type=jnp.float32) * vg
            ).reshape(TL, K, D).sum(axis=1)
    out += (jnp.dot(p_g.reshape(TL * G, H), St,
                    preferred_element_type=jnp.float32) * gv_exp
            ).reshape(TL, G, D).sum(axis=1)

    o_ref[0] = (jnp.dot(out, wout_ref[...], preferred_element_type=jnp.float32)
                + bout_ref[0:1, :])


@functools.cache
def _make_sc_gather():
    return pl.kernel(
        _sc_gather_body,
        mesh=plsc.VectorSubcoreMesh(core_axis_name="c", subcore_axis_name="s"),
        out_type=[jax.ShapeDtypeStruct((B * L * K, D), jnp.float32)] * 2,
        scratch_types=[
            pltpu.VMEM((NCHUNK, IDX_CH), jnp.int32),
            pltpu.VMEM((NBUF, IDX_CH, D), jnp.float32),
            pltpu.VMEM((NBUF, IDX_CH, D), jnp.float32),
            pltpu.SemaphoreType.DMA,
            pltpu.SemaphoreType.DMA,
            pltpu.SemaphoreType.DMA,
            pltpu.SemaphoreType.DMA,
        ],
    )


def kernel(spatial, topk_indices, rpe, self_rpe, distances, global_latents,
           Wq, Wk, Wv, Wout, b_out, log_sigma, global_bias):
    xall = jnp.concatenate(
        [spatial, global_latents,
         jnp.zeros((B, LP - L - G, D), spatial.dtype)], axis=1)

    q_all, kp, vp = pl.pallas_call(
        _proj_body,
        grid=(B, LP // TLA),
        in_specs=[
            pl.BlockSpec((1, TLA, D), lambda b, i: (b, i, 0)),
            pl.BlockSpec((D, D), lambda b, i: (0, 0)),
            pl.BlockSpec((D, D), lambda b, i: (0, 0)),
            pl.BlockSpec((D, D), lambda b, i: (0, 0)),
        ],
        out_specs=[
            pl.BlockSpec((1, TLA, D), lambda b, i: (b, i, 0)),
            pl.BlockSpec((1, TLA, D), lambda b, i: (b, i, 0)),
            pl.BlockSpec((1, TLA, D), lambda b, i: (b, i, 0)),
        ],
        out_shape=[jax.ShapeDtypeStruct((B, LP, D), jnp.float32)] * 3,
    )(xall, Wq, Wk, Wv)

    q = q_all[:, :L]
    idx_chunks = topk_indices.astype(jnp.int32).reshape(B * L // CH, IDX_CH)
    kg_flat, vg_flat = _make_sc_gather()(
        kp.reshape(B * LP, D), vp.reshape(B * LP, D), idx_chunks)
    kg = kg_flat.reshape(B, L * K, D)
    vg = vg_flat.reshape(B, L * K, D)

    params = jnp.zeros((8, 128), jnp.float32)
    params = params.at[0, :H].set(-0.5 * jnp.exp(-2.0 * log_sigma))
    params = params.at[1, 0].set(global_bias)
    bout8 = jnp.broadcast_to(b_out[None, :], (8, D))
    gk = kp[:, L:L + G]
    gv = vp[:, L:L + G]

    out = pl.pallas_call(
        _attn_body,
        grid=(B, L // TL),
        in_specs=[
            pl.BlockSpec((1, TL, D), lambda b, i: (b, i, 0)),
            pl.BlockSpec((1, TL * K, D), lambda b, i: (b, i, 0)),
            pl.BlockSpec((1, TL * K, D), lambda b, i: (b, i, 0)),
            pl.BlockSpec((1, TL, D), lambda b, i: (b, i, 0)),
            pl.BlockSpec((1, TL, D), lambda b, i: (b, i, 0)),
            pl.BlockSpec((1, G, D), lambda b, i: (b, 0, 0)),
            pl.BlockSpec((1, G, D), lambda b, i: (b, 0, 0)),
            pl.BlockSpec((1, TL, K), lambda b, i: (b, i, 0)),
            pl.BlockSpec((8, 128), lambda b, i: (0, 0)),
            pl.BlockSpec((D, D), lambda b, i: (0, 0)),
            pl.BlockSpec((8, D), lambda b, i: (0, 0)),
        ],
        out_specs=pl.BlockSpec((1, TL, D), lambda b, i: (b, i, 0)),
        out_shape=jax.ShapeDtypeStruct((B, L, D), jnp.float32),
    )(q, kg, vg, kp[:, :L], vp[:, :L], gk, gv, distances, params, Wout, bout8)
    return out


# speedup vs baseline: 3.8872x; 1.0336x over previous
import functools
import jax
import jax.numpy as jnp
from jax import lax
from jax.experimental import pallas as pl
from jax.experimental.pallas import tpu as pltpu
from jax.experimental.pallas import tpu_sc as plsc

H = 12
DH = 64
L = 2048
D = 768
K = 16
G = 8
B = 2
LP = 2176
TLA = 128
TL = 128
SCALE = DH ** -0.5

NW = 32
ROWS_PER_W = B * L // NW
CH = 2
IDX_CH = CH * K
NCHUNK = ROWS_PER_W // CH
NBUF = 2


def _proj_body(x_ref, wq_ref, wk_ref, wv_ref, q_ref, k_ref, v_ref):
    x = x_ref[0]
    q_ref[0] = jnp.dot(x, wq_ref[...], preferred_element_type=jnp.float32)
    k_ref[0] = jnp.dot(x, wk_ref[...], preferred_element_type=jnp.float32)
    v_ref[0] = jnp.dot(x, wv_ref[...], preferred_element_type=jnp.float32)


def _sc_gather_body(kp_hbm, vp_hbm, idx_hbm, kg_hbm, vg_hbm,
                    idx_v, kbuf, vbuf, ks0, ks1, vs0, vs1):
    ksem = (ks0, ks1)
    vsem = (vs0, vs1)
    wid = lax.axis_index("s") * 2 + lax.axis_index("c")
    row0 = wid * ROWS_PER_W
    boff = (row0 // L) * LP
    chunk0 = wid * NCHUNK

    pltpu.sync_copy(idx_hbm.at[pl.ds(chunk0, NCHUNK)], idx_v)

    def adjust(c, carry):
        for h in range(IDX_CH // 16):
            sl = pl.ds(h * 16, 16)
            idx_v[c, sl] = idx_v[c, sl] + boff
        return carry

    lax.fori_loop(0, NCHUNK, adjust, 0)

    def fire(c, s):
        pltpu.async_copy(kp_hbm.at[idx_v.at[c]], kbuf.at[s], ksem[s])
        pltpu.async_copy(vp_hbm.at[idx_v.at[c]], vbuf.at[s], vsem[s])

    def drain(c, s):
        pltpu.make_async_copy(kp_hbm.at[idx_v.at[c]], kbuf.at[s],
                              ksem[s]).wait()
        pltpu.make_async_copy(vp_hbm.at[idx_v.at[c]], vbuf.at[s],
                              vsem[s]).wait()
        out_off = (chunk0 + c) * IDX_CH
        pltpu.sync_copy(kbuf.at[s], kg_hbm.at[pl.ds(out_off, IDX_CH)])
        pltpu.sync_copy(vbuf.at[s], vg_hbm.at[pl.ds(out_off, IDX_CH)])

    for s in range(NBUF):
        fire(s, s)

    def group(gi, carry):
        c = gi * NBUF
        for s in range(NBUF):
            drain(c + s, s)
            fire(c + NBUF + s, s)
        return carry

    lax.fori_loop(0, (NCHUNK - NBUF) // NBUF, group, 0)
    for s in range(NBUF):
        drain(NCHUNK - NBUF + s, s)


def _attn_body(q_ref, kg_ref, vg_ref, kself_ref, vself_ref, gk_ref, gv_ref,
               dist_ref, par_ref, wout_ref, bout_ref, o_ref):
    q = q_ref[0] * SCALE
    kg = kg_ref[0]
    vg = vg_ref[0]

    d_over = lax.broadcasted_iota(jnp.int32, (D, H), 0) // DH
    h_ids = lax.broadcasted_iota(jnp.int32, (D, H), 1)
    S = (d_over == h_ids).astype(jnp.float32)
    d_over_t = lax.broadcasted_iota(jnp.int32, (H, D), 1) // DH
    h_ids_t = lax.broadcasted_iota(jnp.int32, (H, D), 0)
    St = (d_over_t == h_ids_t).astype(jnp.float32)

    kself = kself_ref[0]
    vself = vself_ref[0]
    gk = gk_ref[0]
    gv = gv_ref[0]

    qexp_n = jnp.broadcast_to(q[:, None, :], (TL, K, D)).reshape(TL * K, D)
    qexp_g = jnp.broadcast_to(q[:, None, :], (TL, G, D)).reshape(TL * G, D)
    gk_exp = jnp.broadcast_to(gk[None, :, :], (TL, G, D)).reshape(TL * G, D)
    gv_exp = jnp.broadcast_to(gv[None, :, :], (TL, G, D)).reshape(TL * G, D)

    s_n = jnp.dot(qexp_n * kg, S, preferred_element_type=jnp.float32)
    s_s = jnp.dot(q * kself, S, preferred_element_type=jnp.float32)
    s_g = jnp.dot(qexp_g * gk_exp, S, preferred_element_type=jnp.float32)

    inv2s = par_ref[0:1, 0:H].reshape(1, 1, H)
    gbias = par_ref[1:2, 0:1].reshape(1, 1, 1)
    dist = dist_ref[0]
    s_n3 = s_n.reshape(TL, K, H) + (dist * dist)[:, :, None] * inv2s
    s_g3 = s_g.reshape(TL, G, H) + gbias

    m = jnp.maximum(jnp.maximum(s_s, s_n3.max(axis=1)), s_g3.max(axis=1))
    e_s = jnp.exp(s_s - m)
    e_n = jnp.exp(s_n3 - m[:, None, :])
    e_g = jnp.exp(s_g3 - m[:, None, :])
    inv_den = 1.0 / (e_s + e_n.sum(axis=1) + e_g.sum(axis=1))
    p_s = e_s * inv_den
    p_n = e_n * inv_den[:, None, :]
    p_g = e_g * inv_den[:, None, :]

    out = jnp.dot(p_s, St, preferred_element_type=jnp.float32) * vself
    out += (jnp.dot(p_n.reshape(TL * K, H), St,
                    preferred_element_type=jnp.float32) * vg
            ).reshape(TL, K, D).sum(axis=1)
    out += (jnp.dot(p_g.reshape(TL * G, H), St,
                    preferred_element_type=jnp.float32) * gv_exp
            ).reshape(TL, G, D).sum(axis=1)

    o_ref[0] = (jnp.dot(out, wout_ref[...], preferred_element_type=jnp.float32)
                + bout_ref[0:1, :])


@functools.cache
def _make_sc_gather():
    return pl.kernel(
        _sc_gather_body,
        mesh=plsc.VectorSubcoreMesh(core_axis_name="c", subcore_axis_name="s"),
        out_type=[jax.ShapeDtypeStruct((B * L * K, D), jnp.float32)] * 2,
        scratch_types=[
            pltpu.VMEM((NCHUNK, IDX_CH), jnp.int32),
            pltpu.VMEM((NBUF, IDX_CH, D), jnp.float32),
            pltpu.VMEM((NBUF, IDX_CH, D), jnp.float32),
            pltpu.SemaphoreType.DMA,
            pltpu.SemaphoreType.DMA,
            pltpu.SemaphoreType.DMA,
            pltpu.SemaphoreType.DMA,
        ],
    )


def kernel(spatial, topk_indices, rpe, self_rpe, distances, global_latents,
           Wq, Wk, Wv, Wout, b_out, log_sigma, global_bias):
    xall = jnp.concatenate(
        [spatial, global_latents,
         jnp.zeros((B, LP - L - G, D), spatial.dtype)], axis=1)

    q_all, kp, vp = pl.pallas_call(
        _proj_body,
        grid=(B, LP // TLA),
        in_specs=[
            pl.BlockSpec((1, TLA, D), lambda b, i: (b, i, 0)),
            pl.BlockSpec((D, D), lambda b, i: (0, 0)),
            pl.BlockSpec((D, D), lambda b, i: (0, 0)),
            pl.BlockSpec((D, D), lambda b, i: (0, 0)),
        ],
        out_specs=[
            pl.BlockSpec((1, TLA, D), lambda b, i: (b, i, 0)),
            pl.BlockSpec((1, TLA, D), lambda b, i: (b, i, 0)),
            pl.BlockSpec((1, TLA, D), lambda b, i: (b, i, 0)),
        ],
        out_shape=[jax.ShapeDtypeStruct((B, LP, D), jnp.float32)] * 3,
    )(xall, Wq, Wk, Wv)

    q = q_all[:, :L]
    idx_chunks = topk_indices.astype(jnp.int32).reshape(B * L // CH, IDX_CH)
    kg_flat, vg_flat = _make_sc_gather()(
        kp.reshape(B * LP, D), vp.reshape(B * LP, D), idx_chunks)
    kg = kg_flat.reshape(B, L * K, D)
    vg = vg_flat.reshape(B, L * K, D)

    params = jnp.zeros((8, 128), jnp.float32)
    params = params.at[0, :H].set(-0.5 * jnp.exp(-2.0 * log_sigma))
    params = params.at[1, 0].set(global_bias)
    bout8 = jnp.broadcast_to(b_out[None, :], (8, D))
    gk = kp[:, L:L + G]
    gv = vp[:, L:L + G]

    out = pl.pallas_call(
        _attn_body,
        grid=(B, L // TL),
        in_specs=[
            pl.BlockSpec((1, TL, D), lambda b, i: (b, i, 0)),
            pl.BlockSpec((1, TL * K, D), lambda b, i: (b, i, 0)),
            pl.BlockSpec((1, TL * K, D), lambda b, i: (b, i, 0)),
            pl.BlockSpec((1, TL, D), lambda b, i: (b, i, 0)),
            pl.BlockSpec((1, TL, D), lambda b, i: (b, i, 0)),
            pl.BlockSpec((1, G, D), lambda b, i: (b, 0, 0)),
            pl.BlockSpec((1, G, D), lambda b, i: (b, 0, 0)),
            pl.BlockSpec((1, TL, K), lambda b, i: (b, i, 0)),
            pl.BlockSpec((8, 128), lambda b, i: (0, 0)),
            pl.BlockSpec((D, D), lambda b, i: (0, 0)),
            pl.BlockSpec((8, D), lambda b, i: (0, 0)),
        ],
        out_specs=pl.BlockSpec((1, TL, D), lambda b, i: (b, i, 0)),
        out_shape=jax.ShapeDtypeStruct((B, L, D), jnp.float32),
    )(q, kg, vg, kp[:, :L], vp[:, :L], gk, gv, distances, params, Wout, bout8)
    return out
